# baseline (speedup 1.0000x reference)
import functools

import jax
import jax.numpy as jnp
from jax import lax
from jax.experimental import pallas as pl
from jax.experimental.pallas import tpu as pltpu
from jax.experimental.pallas import tpu_sc as plsc

_B, _S, _D = 4096, 200, 64
_ROWS = _S * _D
_NW = 32
_RPW = _ROWS // _NW
_CR = 4
_NCH = _RPW // _CR
_NBUF = 4
_L = 16


def _sc_body(x_hbm, pos_hbm, out_hbm, pos_v, b0, b1, b2, b3,
             i0, i1, i2, i3, o0, o1, o2, o3):
    bufs = (b0, b1, b2, b3)
    isems = (i0, i1, i2, i3)
    osems = (o0, o1, o2, o3)
    wid = lax.axis_index("s") * 2 + lax.axis_index("c")
    row0 = wid * _RPW
    pltpu.sync_copy(pos_hbm.at[pl.ds(row0 * _L, _RPW * _L)], pos_v)

    def xslice(c):
        r = row0 + c * _CR
        return x_hbm.at[r // _D, pl.ds(r % _D, _CR), :]

    def oslice(c):
        r = row0 + c * _CR
        return out_hbm.at[r // _D, pl.ds(r % _D, _CR), :]

    def start_in(c, k):
        pltpu.async_copy(xslice(c), bufs[k], isems[k])

    def wait_in(c, k):
        pltpu.make_async_copy(xslice(c), bufs[k], isems[k]).wait()

    def start_out(c, k):
        pltpu.async_copy(bufs[k], oslice(c), osems[k])

    def wait_out(c, k):
        pltpu.make_async_copy(bufs[k], oslice(c), osems[k]).wait()

    def compute(c, k):
        buf = bufs[k]

        def per_row(rh, carry):
            pv = pos_v[pl.ds((_CR * c + rh) * _L, _L)]

            def per_jo(jo, carry2):
                for ji in range(16):
                    sl = pl.ds(jo * 256 + ji * _L, _L)
                    buf[rh, sl] = buf[rh, sl] + pv
                return carry2

            lax.fori_loop(0, _B // 256, per_jo, 0)
            return carry

        lax.fori_loop(0, _CR, per_row, 0)

    def step(c, k):
        wait_in(c, k)
        compute(c, k)
        start_out(c, k)

    start_in(0, 0)
    start_in(1, 1)

    def outer(o, carry):
        for b in range(_NBUF):
            c = _NBUF * o + b
            step(c, b)
            k2 = (b + 2) % _NBUF

            @pl.when(c >= 2)
            def _():
                wait_out(c - 2, k2)

            start_in(c + 2, k2)
        return carry

    lax.fori_loop(0, (_NCH - _NBUF) // _NBUF, outer, 0)

    for c in range(_NCH - _NBUF, _NCH):
        k = c % _NBUF
        step(c, k)
        if c + 2 < _NCH:
            wait_out(c - 2, (c + 2) % _NBUF)
            start_in(c + 2, (c + 2) % _NBUF)
    for c in range(_NCH - _NBUF, _NCH):
        wait_out(c, c % _NBUF)


def kernel(x, pos_emb):
    xt = jnp.transpose(x, (1, 2, 0))
    pos16 = jnp.broadcast_to(
        pos_emb.reshape(_ROWS)[:, None], (_ROWS, _L)
    ).reshape(_ROWS * _L)

    mesh = plsc.VectorSubcoreMesh(core_axis_name="c", subcore_axis_name="s")
    sc_add = functools.partial(
        pl.kernel,
        mesh=mesh,
        out_type=jax.ShapeDtypeStruct((_S, _D, _B), jnp.float32),
        scratch_types=(
            [pltpu.VMEM((_RPW * _L,), jnp.float32)]
            + [pltpu.VMEM((_CR, _B), jnp.float32) for _ in range(_NBUF)]
            + [pltpu.SemaphoreType.DMA for _ in range(2 * _NBUF)]
        ),
    )(_sc_body)
    out = sc_add(xt, pos16)
    return jnp.transpose(out, (2, 0, 1))


# speedup vs baseline: 1.0775x; 1.0775x over previous
import functools

import jax
import jax.numpy as jnp
from jax import lax
from jax.experimental import pallas as pl
from jax.experimental.pallas import tpu as pltpu
from jax.experimental.pallas import tpu_sc as plsc

_B, _S, _D = 4096, 200, 64
_ROWS = _S * _D
_NW = 32
_RPW = _ROWS // _NW
_CR = 8
_NCH = _RPW // _CR
_NBUF = 3
_L = 16


def _sc_body(x_hbm, pos_hbm, out_hbm, pos_v, b0, b1, b2,
             i0, i1, i2, o0, o1, o2):
    bufs = (b0, b1, b2)
    isems = (i0, i1, i2)
    osems = (o0, o1, o2)
    wid = lax.axis_index("s") * 2 + lax.axis_index("c")
    row0 = wid * _RPW
    pltpu.sync_copy(pos_hbm.at[pl.ds(row0 * _L, _RPW * _L)], pos_v)

    def xslice(c):
        r = row0 + c * _CR
        return x_hbm.at[r // _D, pl.ds(r % _D, _CR), :]

    def oslice(c):
        r = row0 + c * _CR
        return out_hbm.at[r // _D, pl.ds(r % _D, _CR), :]

    def start_in(c, k):
        pltpu.async_copy(xslice(c), bufs[k], isems[k])

    def wait_in(c, k):
        pltpu.make_async_copy(xslice(c), bufs[k], isems[k]).wait()

    def start_out(c, k):
        pltpu.async_copy(bufs[k], oslice(c), osems[k])

    def wait_out(c, k):
        pltpu.make_async_copy(bufs[k], oslice(c), osems[k]).wait()

    def compute(c, k):
        buf = bufs[k]

        def per_row(rh, carry):
            pv = pos_v[pl.ds((_CR * c + rh) * _L, _L)]

            def per_jo(jo, carry2):
                for ji in range(16):
                    sl = pl.ds(jo * 256 + ji * _L, _L)
                    buf[rh, sl] = buf[rh, sl] + pv
                return carry2

            lax.fori_loop(0, _B // 256, per_jo, 0)
            return carry

        lax.fori_loop(0, _CR, per_row, 0)

    def step(c, k):
        wait_in(c, k)
        compute(c, k)
        start_out(c, k)

    start_in(0, 0)
    start_in(1, 1)

    def outer(o, carry):
        for b in range(_NBUF):
            c = _NBUF * o + b
            step(c, b)
            k2 = (b + 2) % _NBUF

            @pl.when(c >= 1)
            def _():
                wait_out(c - 1, k2)

            start_in(c + 2, k2)
        return carry

    lax.fori_loop(0, (_NCH - 2) // _NBUF, outer, 0)

    for c in range(_NCH - 2, _NCH):
        step(c, c % _NBUF)
    for c in range(_NCH - _NBUF, _NCH):
        wait_out(c, c % _NBUF)


def kernel(x, pos_emb):
    xt = jnp.transpose(x, (1, 2, 0))
    pos16 = jnp.broadcast_to(
        pos_emb.reshape(_ROWS)[:, None], (_ROWS, _L)
    ).reshape(_ROWS * _L)

    mesh = plsc.VectorSubcoreMesh(core_axis_name="c", subcore_axis_name="s")
    sc_add = functools.partial(
        pl.kernel,
        mesh=mesh,
        out_type=jax.ShapeDtypeStruct((_S, _D, _B), jnp.float32),
        scratch_types=(
            [pltpu.VMEM((_RPW * _L,), jnp.float32)]
            + [pltpu.VMEM((_CR, _B), jnp.float32) for _ in range(_NBUF)]
            + [pltpu.SemaphoreType.DMA for _ in range(2 * _NBUF)]
        ),
    )(_sc_body)
    out = sc_add(xt, pos16)
    return jnp.transpose(out, (2, 0, 1))
